# baseline (speedup 1.0000x reference)
import functools
import math

import jax
import jax.numpy as jnp
from jax.experimental import pallas as pl
from jax.experimental.pallas import tpu as pltpu
from jax.experimental.pallas import tpu_sc as plsc

K = 20
NUM_CLASSES = 3
S_SCALE = 30.0
M_MARGIN = 0.4
LN_EPS = 1e-5

ROWS = 256


def _ln(h, g, be):
    m = jnp.mean(h, axis=-1, keepdims=True)
    v = jnp.mean((h - m) ** 2, axis=-1, keepdims=True)
    return (h - m) / jnp.sqrt(v + LN_EPS) * g + be


def _bdot(a, b):
    return jnp.dot(a.astype(jnp.bfloat16), b.astype(jnp.bfloat16),
                   preferred_element_type=jnp.float32)


def _bdot_nt(a, b):
    return jax.lax.dot_general(
        a.astype(jnp.bfloat16), b.astype(jnp.bfloat16),
        (((1,), (1,)), ((), ())), preferred_element_type=jnp.float32)


KROWS = 128
CHUNK = 128


def _pair_min(bv, bi, cv, ci):
    lt = (cv < bv) | ((cv == bv) & (ci < bi))
    return jnp.where(lt, cv, bv), jnp.where(lt, ci, bi)


def _knn_body(feat_r_ref, feat_ref, sqc_ref, sqr_ref, batch_r_ref, batch_ref,
              idx_ref, d2_ref):
    feat_r = feat_r_ref[...]
    feat = feat_ref[...]
    n = feat.shape[0]
    rows = feat_r.shape[0]

    xx = _bdot_nt(feat_r, feat)
    d2 = sqc_ref[...] + sqr_ref[...] - 2.0 * xx

    b_r = batch_r_ref[0, :].reshape(rows, 1)
    b_all = batch_ref[0, :].reshape(1, n)
    d2_ref[...] = jnp.where(b_r != b_all, 1e10, d2)

    nchunks = n // CHUNK
    big_i = jnp.int32(2**30)

    cols = []
    amin = jnp.full((rows, 1), -1, jnp.int32)
    for k in range(K):
        def chunk_body(c, carry):
            bv, bi = carry
            off = c * CHUNK
            v = d2_ref[:, pl.ds(off, CHUNK)]
            ci = jax.lax.broadcasted_iota(jnp.int32, (rows, CHUNK), 1) + off
            if k > 0:
                v = jnp.where(ci == amin, 1e10, v)
                d2_ref[:, pl.ds(off, CHUNK)] = v
            return _pair_min(bv, bi, v, ci)

        bv = jnp.full((rows, CHUNK), jnp.inf, jnp.float32)
        bi = jnp.full((rows, CHUNK), big_i, jnp.int32)
        bv, bi = jax.lax.fori_loop(0, nchunks, chunk_body, (bv, bi))
        w = CHUNK // 2
        while w >= 1:
            bv, bi = _pair_min(bv[:, :w], bi[:, :w],
                               bv[:, w:2 * w], bi[:, w:2 * w])
            w //= 2
        amin = bi
        cols.append(amin)
    idx_ref[...] = jnp.concatenate(cols, axis=1)


def _knn(feat, sqcol, sqrow, batch2d):
    n, d = feat.shape
    return pl.pallas_call(
        _knn_body,
        grid=(n // KROWS,),
        in_specs=[
            pl.BlockSpec((KROWS, d), lambda i: (i, 0)),
            pl.BlockSpec((n, d), lambda i: (0, 0)),
            pl.BlockSpec((KROWS, 1), lambda i: (i, 0)),
            pl.BlockSpec((1, n), lambda i: (0, 0)),
            pl.BlockSpec((1, KROWS), lambda i: (0, i)),
            pl.BlockSpec((1, n), lambda i: (0, 0)),
        ],
        out_specs=pl.BlockSpec((KROWS, K), lambda i: (i, 0)),
        out_shape=jax.ShapeDtypeStruct((n, K), jnp.int32),
        scratch_shapes=[pltpu.VMEM((KROWS, n), jnp.float32)],
    )(feat, feat, sqcol, sqrow, batch2d, batch2d)


GATHER_WINDOW = 128


def _sc_gather(table, idx_flat):
    num_idx = idx_flat.shape[0]
    h = table.shape[1]
    idx2d = idx_flat.reshape(1, num_idx)
    mesh = plsc.VectorSubcoreMesh(core_axis_name="core",
                                  subcore_axis_name="subcore")

    @pl.kernel(out_type=jax.ShapeDtypeStruct((num_idx, h), table.dtype),
               mesh=mesh)
    def gather_kernel(t_hbm, i_hbm, o_hbm):
        def body(i_vmem, o_vmem):
            pltpu.sync_copy(t_hbm.at[i_vmem.at[0]], o_vmem)

        pltpu.emit_pipeline(
            body,
            grid=(num_idx // GATHER_WINDOW,),
            in_specs=[pl.BlockSpec((1, GATHER_WINDOW),
                                   index_map=lambda i: (0, i))],
            out_specs=[pl.BlockSpec((GATHER_WINDOW, h),
                                    index_map=lambda i: (i, 0))],
            core_axis_name=("core", "subcore"),
            dimension_semantics=(pltpu.PARALLEL,),
        )(i_hbm, o_hbm)

    return gather_kernel(table, idx2d)


def _edge_body(feat_r_ref, xj_ref, w1t_ref, b1_ref, g1_ref, be1_ref,
               w2t_ref, b2_ref, g2_ref, be2_ref, out_ref):
    rows = feat_r_ref.shape[0]
    d = feat_r_ref.shape[1]
    h1 = w1t_ref.shape[1]
    h2 = out_ref.shape[1]
    xi = feat_r_ref[...]
    xj = xj_ref[...][:, :d].reshape(rows, K, d)
    xi3 = jnp.broadcast_to(xi[:, None, :], (rows, K, d))
    e = jnp.concatenate([xi3, xj - xi3], axis=2).reshape(rows * K, 2 * d)
    z1 = _bdot(e, w1t_ref[...]) + b1_ref[...]
    a1 = jax.nn.relu(_ln(z1, g1_ref[0, :], be1_ref[0, :]))
    z2 = _bdot(a1, w2t_ref[...]) + b2_ref[...]
    a2 = jax.nn.relu(_ln(z2, g2_ref[0, :], be2_ref[0, :]))
    out_ref[...] = jnp.max(a2.reshape(rows, K, h2), axis=1)


def _edge_mlp(feat, xjg, w1t, b1, g1, be1, w2t, b2, g2, be2):
    n, d = feat.shape
    hp = xjg.shape[1]
    h1 = w1t.shape[1]
    h2 = w2t.shape[1]
    return pl.pallas_call(
        _edge_body,
        grid=(n // ROWS,),
        in_specs=[
            pl.BlockSpec((ROWS, d), lambda i: (i, 0)),
            pl.BlockSpec((ROWS * K, hp), lambda i: (i, 0)),
            pl.BlockSpec((2 * d, h1), lambda i: (0, 0)),
            pl.BlockSpec((1, h1), lambda i: (0, 0)),
            pl.BlockSpec((1, h1), lambda i: (0, 0)),
            pl.BlockSpec((1, h1), lambda i: (0, 0)),
            pl.BlockSpec((h1, h2), lambda i: (0, 0)),
            pl.BlockSpec((1, h2), lambda i: (0, 0)),
            pl.BlockSpec((1, h2), lambda i: (0, 0)),
            pl.BlockSpec((1, h2), lambda i: (0, 0)),
        ],
        out_specs=pl.BlockSpec((ROWS, h2), lambda i: (i, 0)),
        out_shape=jax.ShapeDtypeStruct((n, h2), jnp.float32),
    )(feat, xjg, w1t, b1, g1, be1, w2t, b2, g2, be2)


def _edge_conv(feat, batch2d, p):
    n, d = feat.shape
    sq = jnp.sum(feat * feat, axis=1)
    idx = _knn(feat, sq.reshape(n, 1), sq.reshape(1, n), batch2d)
    hp = ((d + 127) // 128) * 128
    table = jnp.pad(feat, ((0, 0), (0, hp - d)))
    xjg = _sc_gather(table, idx.reshape(-1))
    h1 = p['l1']['W'].shape[0]
    h2 = p['l2']['W'].shape[0]
    return _edge_mlp(
        feat, xjg,
        p['l1']['W'].T, p['l1']['b'].reshape(1, h1),
        p['l1']['g'].reshape(1, h1), p['l1']['be'].reshape(1, h1),
        p['l2']['W'].T, p['l2']['b'].reshape(1, h2),
        p['l2']['g'].reshape(1, h2), p['l2']['be'].reshape(1, h2))


def _global_body(nb, x1_ref, x2_ref, x3_ref, bcol_ref, w1t_ref, b1_ref,
                 g1_ref, be1_ref, w2t_ref, b2_ref, g2_ref, be2_ref, g_ref):
    bcol = bcol_ref[...]
    rows = []
    for b in range(nb):
        mask = bcol == b
        pieces = []
        for xr in (x1_ref, x2_ref, x3_ref):
            xv = xr[...]
            masked = jnp.where(mask, xv, -jnp.inf)
            pieces.append(jnp.max(masked, axis=0, keepdims=True))
        rows.append(jnp.concatenate(pieces, axis=1))
    pooled = jnp.concatenate(rows, axis=0)
    z1 = _bdot(pooled, w1t_ref[...]) + b1_ref[...]
    a1 = jax.nn.relu(_ln(z1, g1_ref[0, :], be1_ref[0, :]))
    z2 = _bdot(a1, w2t_ref[...]) + b2_ref[...]
    g_ref[...] = jax.nn.relu(_ln(z2, g2_ref[0, :], be2_ref[0, :]))


def _global_mlp(nb, x1, x2, x3, bcol, gp):
    hg1 = gp['l1']['W'].shape[0]
    hg2 = gp['l2']['W'].shape[0]
    args = (x1, x2, x3, bcol, gp['l1']['W'].T, gp['l1']['b'].reshape(1, hg1),
            gp['l1']['g'].reshape(1, hg1), gp['l1']['be'].reshape(1, hg1),
            gp['l2']['W'].T, gp['l2']['b'].reshape(1, hg2),
            gp['l2']['g'].reshape(1, hg2), gp['l2']['be'].reshape(1, hg2))
    return pl.pallas_call(
        functools.partial(_global_body, nb),
        out_specs=pl.BlockSpec((nb, hg2), lambda: (0, 0)),
        out_shape=jax.ShapeDtypeStruct((nb, hg2), jnp.float32),
    )(*args)


def _head_body(nb, x1_ref, x2_ref, x3_ref, g_ref, bcol_ref, ycol_ref,
               w1t_ref, b1_ref, g1_ref, be1_ref,
               w2t_ref, b2_ref, g2_ref, be2_ref,
               w3t_ref, b3_ref, g3_ref, be3_ref,
               wnt_ref, out_ref):
    rows = x1_ref.shape[0]
    bcol = bcol_ref[...]
    g = g_ref[...]
    gf = jnp.zeros((rows, g.shape[1]), jnp.float32)
    for b in range(nb):
        gf = jnp.where(bcol == b, g[b:b + 1, :], gf)
    combined = jnp.concatenate(
        [x1_ref[...], x2_ref[...], x3_ref[...], gf], axis=1)
    z1 = _bdot(combined, w1t_ref[...]) + b1_ref[...]
    a1 = jax.nn.relu(_ln(z1, g1_ref[0, :], be1_ref[0, :]))
    z2 = _bdot(a1, w2t_ref[...]) + b2_ref[...]
    a2 = jax.nn.relu(_ln(z2, g2_ref[0, :], be2_ref[0, :]))
    z3 = _bdot(a2, w3t_ref[...]) + b3_ref[...]
    emb = _ln(z3, g3_ref[0, :], be3_ref[0, :])

    norm = jnp.sqrt(jnp.sum(emb * emb, axis=1, keepdims=True))
    emb_n = emb / jnp.maximum(norm, 1e-12)
    cos = _bdot(emb_n, wnt_ref[...])
    cos = jnp.clip(cos, -1.0, 1.0)
    sin = jnp.sqrt(jnp.maximum(1.0 - cos * cos, 0.0))
    labels = ycol_ref[...] - 1
    cls = jax.lax.broadcasted_iota(jnp.int32, cos.shape, 1)
    cos_m = math.cos(M_MARGIN)
    sin_m = math.sin(M_MARGIN)
    out = jnp.where(cls == labels, cos * cos_m - sin * sin_m, cos)
    out_ref[...] = out * S_SCALE


def _head(nb, x1, x2, x3, g, bcol, ycol, hp, arc_w):
    n = x1.shape[0]
    h1 = hp['l1']['W'].shape[0]
    h2 = hp['l2']['W'].shape[0]
    h3 = hp['l3']['W'].shape[0]
    wn = arc_w / jnp.maximum(
        jnp.linalg.norm(arc_w, axis=1, keepdims=True), 1e-12)
    full = lambda a: pl.BlockSpec(a.shape, lambda i: tuple(0 for _ in a.shape))
    row_tile = lambda w: pl.BlockSpec((ROWS, w), lambda i: (i, 0))
    args = (x1, x2, x3, g, bcol, ycol,
            hp['l1']['W'].T, hp['l1']['b'].reshape(1, h1),
            hp['l1']['g'].reshape(1, h1), hp['l1']['be'].reshape(1, h1),
            hp['l2']['W'].T, hp['l2']['b'].reshape(1, h2),
            hp['l2']['g'].reshape(1, h2), hp['l2']['be'].reshape(1, h2),
            hp['l3']['W'].T, hp['l3']['b'].reshape(1, h3),
            hp['l3']['g'].reshape(1, h3), hp['l3']['be'].reshape(1, h3),
            wn.T)
    in_specs = [row_tile(x1.shape[1]), row_tile(x2.shape[1]),
                row_tile(x3.shape[1]), full(g),
                pl.BlockSpec((ROWS, 1), lambda i: (i, 0)),
                pl.BlockSpec((ROWS, 1), lambda i: (i, 0))]
    in_specs += [full(a) for a in args[6:]]
    return pl.pallas_call(
        functools.partial(_head_body, nb),
        grid=(n // ROWS,),
        in_specs=in_specs,
        out_specs=pl.BlockSpec((ROWS, NUM_CLASSES), lambda i: (i, 0)),
        out_shape=jax.ShapeDtypeStruct((n, NUM_CLASSES), jnp.float32),
    )(*args)


def kernel(x, batch, y, params):
    n = x.shape[0]
    nb = 4
    batch = batch.astype(jnp.int32)
    batch2d = batch.reshape(1, n)
    bcol = batch.reshape(n, 1)
    ycol = y.astype(jnp.int32).reshape(n, 1)

    x1 = _edge_conv(x, batch2d, params['conv1'])
    x2 = _edge_conv(x1, batch2d, params['conv2'])
    x3 = _edge_conv(x2, batch2d, params['conv3'])
    g = _global_mlp(nb, x1, x2, x3, bcol, params['global'])
    return _head(nb, x1, x2, x3, g, bcol, ycol, params['head'],
                 params['arc_w'])


# speedup vs baseline: 2.1831x; 2.1831x over previous
import functools
import math

import jax
import jax.numpy as jnp
from jax.experimental import pallas as pl
from jax.experimental.pallas import tpu as pltpu
from jax.experimental.pallas import tpu_sc as plsc

K = 20
NUM_CLASSES = 3
S_SCALE = 30.0
M_MARGIN = 0.4
LN_EPS = 1e-5

ROWS = 256


def _ln(h, g, be):
    m = jnp.mean(h, axis=-1, keepdims=True)
    v = jnp.mean((h - m) ** 2, axis=-1, keepdims=True)
    return (h - m) / jnp.sqrt(v + LN_EPS) * g + be


def _bdot(a, b):
    return jnp.dot(a.astype(jnp.bfloat16), b.astype(jnp.bfloat16),
                   preferred_element_type=jnp.float32)


def _bdot_nt(a, b):
    return jax.lax.dot_general(
        a.astype(jnp.bfloat16), b.astype(jnp.bfloat16),
        (((1,), (1,)), ((), ())), preferred_element_type=jnp.float32)


KROWS = 128
CHUNKW = 128
POOL_S = 5


def _knn_body(feat_r_ref, feat_ref, sqc_ref, sqr_ref, batch_r_ref, batch_ref,
              idx_ref):
    feat_r = feat_r_ref[...]
    feat = feat_ref[...]
    n = feat.shape[0]
    rows = feat_r.shape[0]
    nch = n // CHUNKW
    big_i = jnp.int32(2**30)

    xx = _bdot_nt(feat_r, feat)
    d2 = sqc_ref[...] + sqr_ref[...] - 2.0 * xx

    b_r = batch_r_ref[0, :].reshape(rows, 1)
    b_all = batch_ref[0, :].reshape(1, n)
    d2 = jnp.where(b_r != b_all, 1e10, d2)

    d3 = d2.reshape(rows, nch, CHUNKW)
    g3 = (jax.lax.broadcasted_iota(jnp.int32, (rows, nch, CHUNKW), 2)
          + jax.lax.broadcasted_iota(jnp.int32, (rows, nch, CHUNKW), 1)
          * CHUNKW)
    dk = d3
    ms, asel = [], []
    for s in range(POOL_S):
        m = jnp.min(dk, axis=2)
        a = jnp.min(jnp.where(dk <= m[:, :, None], g3, big_i), axis=2)
        ms.append(m)
        asel.append(a)
        if s < POOL_S - 1:
            dk = jnp.where(g3 == a[:, :, None], 1e10, dk)

    ci = jax.lax.broadcasted_iota(jnp.int32, (rows, nch), 1)
    uses = jnp.zeros((rows, nch), jnp.int32)
    cols = []
    for k in range(K):
        av = jnp.full((rows, nch), jnp.inf, jnp.float32)
        ai = jnp.full((rows, nch), big_i, jnp.int32)
        for s in reversed(range(POOL_S)):
            sel = uses == s
            av = jnp.where(sel, ms[s], av)
            ai = jnp.where(sel, asel[s], ai)
        m = jnp.min(av, axis=1, keepdims=True)
        cmin = jnp.min(jnp.where(av <= m, ci, big_i), axis=1, keepdims=True)
        elem = jnp.min(jnp.where(ci == cmin, ai, big_i), axis=1, keepdims=True)
        uses = uses + (ci == cmin).astype(jnp.int32)
        cols.append(elem)
        if k == K - 1:
            v_last, i_last = m, elem
    idx_ref[...] = jnp.concatenate(cols, axis=1)

    iota2 = jax.lax.broadcasted_iota(jnp.int32, (rows, n), 1)
    lex_le = (d2 < v_last) | ((d2 == v_last) & (iota2 <= i_last))
    count = jnp.sum(lex_le.astype(jnp.int32), axis=1, keepdims=True)
    nbad = jnp.max(jnp.abs(count - K))

    @pl.when(nbad != 0)
    def _fallback():
        dd = d2
        cols2 = []
        for _ in range(K):
            mm = jnp.min(dd, axis=1, keepdims=True)
            am = jnp.min(jnp.where(dd <= mm, iota2, big_i), axis=1,
                         keepdims=True)
            cols2.append(am)
            dd = jnp.where(iota2 == am, 1e10, dd)
        idx_ref[...] = jnp.concatenate(cols2, axis=1)


def _knn(feat, sqcol, sqrow, batch2d):
    n, d = feat.shape
    return pl.pallas_call(
        _knn_body,
        grid=(n // KROWS,),
        in_specs=[
            pl.BlockSpec((KROWS, d), lambda i: (i, 0)),
            pl.BlockSpec((n, d), lambda i: (0, 0)),
            pl.BlockSpec((KROWS, 1), lambda i: (i, 0)),
            pl.BlockSpec((1, n), lambda i: (0, 0)),
            pl.BlockSpec((1, KROWS), lambda i: (0, i)),
            pl.BlockSpec((1, n), lambda i: (0, 0)),
        ],
        out_specs=pl.BlockSpec((KROWS, K), lambda i: (i, 0)),
        out_shape=jax.ShapeDtypeStruct((n, K), jnp.int32),
    )(feat, feat, sqcol, sqrow, batch2d, batch2d)


GATHER_WINDOW = 128


def _sc_gather(table, idx_flat):
    num_idx = idx_flat.shape[0]
    h = table.shape[1]
    idx2d = idx_flat.reshape(1, num_idx)
    mesh = plsc.VectorSubcoreMesh(core_axis_name="core",
                                  subcore_axis_name="subcore")

    @pl.kernel(out_type=jax.ShapeDtypeStruct((num_idx, h), table.dtype),
               mesh=mesh)
    def gather_kernel(t_hbm, i_hbm, o_hbm):
        def body(i_vmem, o_vmem):
            pltpu.sync_copy(t_hbm.at[i_vmem.at[0]], o_vmem)

        pltpu.emit_pipeline(
            body,
            grid=(num_idx // GATHER_WINDOW,),
            in_specs=[pl.BlockSpec((1, GATHER_WINDOW),
                                   index_map=lambda i: (0, i))],
            out_specs=[pl.BlockSpec((GATHER_WINDOW, h),
                                    index_map=lambda i: (i, 0))],
            core_axis_name=("core", "subcore"),
            dimension_semantics=(pltpu.PARALLEL,),
        )(i_hbm, o_hbm)

    return gather_kernel(table, idx2d)


def _edge_body(feat_r_ref, xj_ref, w1t_ref, b1_ref, g1_ref, be1_ref,
               w2t_ref, b2_ref, g2_ref, be2_ref, out_ref):
    rows = feat_r_ref.shape[0]
    d = feat_r_ref.shape[1]
    h1 = w1t_ref.shape[1]
    h2 = out_ref.shape[1]
    xi = feat_r_ref[...]
    xj = xj_ref[...][:, :d].reshape(rows, K, d)
    xi3 = jnp.broadcast_to(xi[:, None, :], (rows, K, d))
    e = jnp.concatenate([xi3, xj - xi3], axis=2).reshape(rows * K, 2 * d)
    z1 = _bdot(e, w1t_ref[...]) + b1_ref[...]
    a1 = jax.nn.relu(_ln(z1, g1_ref[0, :], be1_ref[0, :]))
    z2 = _bdot(a1, w2t_ref[...]) + b2_ref[...]
    a2 = jax.nn.relu(_ln(z2, g2_ref[0, :], be2_ref[0, :]))
    out_ref[...] = jnp.max(a2.reshape(rows, K, h2), axis=1)


def _edge_mlp(feat, xjg, w1t, b1, g1, be1, w2t, b2, g2, be2):
    n, d = feat.shape
    hp = xjg.shape[1]
    h1 = w1t.shape[1]
    h2 = w2t.shape[1]
    return pl.pallas_call(
        _edge_body,
        grid=(n // ROWS,),
        in_specs=[
            pl.BlockSpec((ROWS, d), lambda i: (i, 0)),
            pl.BlockSpec((ROWS * K, hp), lambda i: (i, 0)),
            pl.BlockSpec((2 * d, h1), lambda i: (0, 0)),
            pl.BlockSpec((1, h1), lambda i: (0, 0)),
            pl.BlockSpec((1, h1), lambda i: (0, 0)),
            pl.BlockSpec((1, h1), lambda i: (0, 0)),
            pl.BlockSpec((h1, h2), lambda i: (0, 0)),
            pl.BlockSpec((1, h2), lambda i: (0, 0)),
            pl.BlockSpec((1, h2), lambda i: (0, 0)),
            pl.BlockSpec((1, h2), lambda i: (0, 0)),
        ],
        out_specs=pl.BlockSpec((ROWS, h2), lambda i: (i, 0)),
        out_shape=jax.ShapeDtypeStruct((n, h2), jnp.float32),
    )(feat, xjg, w1t, b1, g1, be1, w2t, b2, g2, be2)


def _edge_conv(feat, batch2d, p):
    n, d = feat.shape
    sq = jnp.sum(feat * feat, axis=1)
    idx = _knn(feat, sq.reshape(n, 1), sq.reshape(1, n), batch2d)
    hp = ((d + 127) // 128) * 128
    table = jnp.pad(feat, ((0, 0), (0, hp - d)))
    xjg = _sc_gather(table, idx.reshape(-1))
    h1 = p['l1']['W'].shape[0]
    h2 = p['l2']['W'].shape[0]
    return _edge_mlp(
        feat, xjg,
        p['l1']['W'].T, p['l1']['b'].reshape(1, h1),
        p['l1']['g'].reshape(1, h1), p['l1']['be'].reshape(1, h1),
        p['l2']['W'].T, p['l2']['b'].reshape(1, h2),
        p['l2']['g'].reshape(1, h2), p['l2']['be'].reshape(1, h2))


def _global_body(nb, x1_ref, x2_ref, x3_ref, bcol_ref, w1t_ref, b1_ref,
                 g1_ref, be1_ref, w2t_ref, b2_ref, g2_ref, be2_ref, g_ref):
    bcol = bcol_ref[...]
    rows = []
    for b in range(nb):
        mask = bcol == b
        pieces = []
        for xr in (x1_ref, x2_ref, x3_ref):
            xv = xr[...]
            masked = jnp.where(mask, xv, -jnp.inf)
            pieces.append(jnp.max(masked, axis=0, keepdims=True))
        rows.append(jnp.concatenate(pieces, axis=1))
    pooled = jnp.concatenate(rows, axis=0)
    z1 = _bdot(pooled, w1t_ref[...]) + b1_ref[...]
    a1 = jax.nn.relu(_ln(z1, g1_ref[0, :], be1_ref[0, :]))
    z2 = _bdot(a1, w2t_ref[...]) + b2_ref[...]
    g_ref[...] = jax.nn.relu(_ln(z2, g2_ref[0, :], be2_ref[0, :]))


def _global_mlp(nb, x1, x2, x3, bcol, gp):
    hg1 = gp['l1']['W'].shape[0]
    hg2 = gp['l2']['W'].shape[0]
    args = (x1, x2, x3, bcol, gp['l1']['W'].T, gp['l1']['b'].reshape(1, hg1),
            gp['l1']['g'].reshape(1, hg1), gp['l1']['be'].reshape(1, hg1),
            gp['l2']['W'].T, gp['l2']['b'].reshape(1, hg2),
            gp['l2']['g'].reshape(1, hg2), gp['l2']['be'].reshape(1, hg2))
    return pl.pallas_call(
        functools.partial(_global_body, nb),
        out_specs=pl.BlockSpec((nb, hg2), lambda: (0, 0)),
        out_shape=jax.ShapeDtypeStruct((nb, hg2), jnp.float32),
    )(*args)


def _head_body(nb, x1_ref, x2_ref, x3_ref, g_ref, bcol_ref, ycol_ref,
               w1t_ref, b1_ref, g1_ref, be1_ref,
               w2t_ref, b2_ref, g2_ref, be2_ref,
               w3t_ref, b3_ref, g3_ref, be3_ref,
               wnt_ref, out_ref):
    rows = x1_ref.shape[0]
    bcol = bcol_ref[...]
    g = g_ref[...]
    gf = jnp.zeros((rows, g.shape[1]), jnp.float32)
    for b in range(nb):
        gf = jnp.where(bcol == b, g[b:b + 1, :], gf)
    combined = jnp.concatenate(
        [x1_ref[...], x2_ref[...], x3_ref[...], gf], axis=1)
    z1 = _bdot(combined, w1t_ref[...]) + b1_ref[...]
    a1 = jax.nn.relu(_ln(z1, g1_ref[0, :], be1_ref[0, :]))
    z2 = _bdot(a1, w2t_ref[...]) + b2_ref[...]
    a2 = jax.nn.relu(_ln(z2, g2_ref[0, :], be2_ref[0, :]))
    z3 = _bdot(a2, w3t_ref[...]) + b3_ref[...]
    emb = _ln(z3, g3_ref[0, :], be3_ref[0, :])

    norm = jnp.sqrt(jnp.sum(emb * emb, axis=1, keepdims=True))
    emb_n = emb / jnp.maximum(norm, 1e-12)
    cos = _bdot(emb_n, wnt_ref[...])
    cos = jnp.clip(cos, -1.0, 1.0)
    sin = jnp.sqrt(jnp.maximum(1.0 - cos * cos, 0.0))
    labels = ycol_ref[...] - 1
    cls = jax.lax.broadcasted_iota(jnp.int32, cos.shape, 1)
    cos_m = math.cos(M_MARGIN)
    sin_m = math.sin(M_MARGIN)
    out = jnp.where(cls == labels, cos * cos_m - sin * sin_m, cos)
    out_ref[...] = out * S_SCALE


def _head(nb, x1, x2, x3, g, bcol, ycol, hp, arc_w):
    n = x1.shape[0]
    h1 = hp['l1']['W'].shape[0]
    h2 = hp['l2']['W'].shape[0]
    h3 = hp['l3']['W'].shape[0]
    wn = arc_w / jnp.maximum(
        jnp.linalg.norm(arc_w, axis=1, keepdims=True), 1e-12)
    full = lambda a: pl.BlockSpec(a.shape, lambda i: tuple(0 for _ in a.shape))
    row_tile = lambda w: pl.BlockSpec((ROWS, w), lambda i: (i, 0))
    args = (x1, x2, x3, g, bcol, ycol,
            hp['l1']['W'].T, hp['l1']['b'].reshape(1, h1),
            hp['l1']['g'].reshape(1, h1), hp['l1']['be'].reshape(1, h1),
            hp['l2']['W'].T, hp['l2']['b'].reshape(1, h2),
            hp['l2']['g'].reshape(1, h2), hp['l2']['be'].reshape(1, h2),
            hp['l3']['W'].T, hp['l3']['b'].reshape(1, h3),
            hp['l3']['g'].reshape(1, h3), hp['l3']['be'].reshape(1, h3),
            wn.T)
    in_specs = [row_tile(x1.shape[1]), row_tile(x2.shape[1]),
                row_tile(x3.shape[1]), full(g),
                pl.BlockSpec((ROWS, 1), lambda i: (i, 0)),
                pl.BlockSpec((ROWS, 1), lambda i: (i, 0))]
    in_specs += [full(a) for a in args[6:]]
    return pl.pallas_call(
        functools.partial(_head_body, nb),
        grid=(n // ROWS,),
        in_specs=in_specs,
        out_specs=pl.BlockSpec((ROWS, NUM_CLASSES), lambda i: (i, 0)),
        out_shape=jax.ShapeDtypeStruct((n, NUM_CLASSES), jnp.float32),
    )(*args)


def kernel(x, batch, y, params):
    n = x.shape[0]
    nb = 4
    batch = batch.astype(jnp.int32)
    batch2d = batch.reshape(1, n)
    bcol = batch.reshape(n, 1)
    ycol = y.astype(jnp.int32).reshape(n, 1)

    x1 = _edge_conv(x, batch2d, params['conv1'])
    x2 = _edge_conv(x1, batch2d, params['conv2'])
    x3 = _edge_conv(x2, batch2d, params['conv3'])
    g = _global_mlp(nb, x1, x2, x3, bcol, params['global'])
    return _head(nb, x1, x2, x3, g, bcol, ycol, params['head'],
                 params['arc_w'])


# speedup vs baseline: 4.3785x; 2.0056x over previous
import functools
import math

import jax
import jax.numpy as jnp
from jax.experimental import pallas as pl
from jax.experimental.pallas import tpu as pltpu
from jax.experimental.pallas import tpu_sc as plsc

K = 20
NUM_CLASSES = 3
S_SCALE = 30.0
M_MARGIN = 0.4
LN_EPS = 1e-5

ROWS = 256


def _ln(h, g, be):
    m = jnp.mean(h, axis=-1, keepdims=True)
    v = jnp.mean((h - m) ** 2, axis=-1, keepdims=True)
    return (h - m) / jnp.sqrt(v + LN_EPS) * g + be


def _bdot(a, b):
    return jnp.dot(a.astype(jnp.bfloat16), b.astype(jnp.bfloat16),
                   preferred_element_type=jnp.float32)


def _bdot_nt(a, b):
    return jax.lax.dot_general(
        a.astype(jnp.bfloat16), b.astype(jnp.bfloat16),
        (((1,), (1,)), ((), ())), preferred_element_type=jnp.float32)


def _knn_body(feat_r_ref, feat_ref, sqc_ref, sqr_ref, batch_r_ref, batch_ref,
              idx_ref):
    feat_r = feat_r_ref[...]
    feat = feat_ref[...]
    n = feat.shape[0]
    rows = feat_r.shape[0]

    xx = _bdot_nt(feat_r, feat)
    d2 = sqc_ref[...] + sqr_ref[...] - 2.0 * xx

    b_r = batch_r_ref[0, :].reshape(rows, 1)
    b_all = batch_ref[0, :].reshape(1, n)
    d2 = jnp.where(b_r != b_all, 1e10, d2)

    iota = jax.lax.broadcasted_iota(jnp.int32, (rows, n), 1)
    cols = []
    for _ in range(K):
        amin = jnp.argmin(d2, axis=1).astype(jnp.int32).reshape(rows, 1)
        cols.append(amin)
        d2 = jnp.where(iota == amin, 1e10, d2)
    idx_ref[...] = jnp.concatenate(cols, axis=1)


def _knn(feat, sqcol, sqrow, batch2d):
    n, d = feat.shape
    return pl.pallas_call(
        _knn_body,
        grid=(n // ROWS,),
        in_specs=[
            pl.BlockSpec((ROWS, d), lambda i: (i, 0)),
            pl.BlockSpec((n, d), lambda i: (0, 0)),
            pl.BlockSpec((ROWS, 1), lambda i: (i, 0)),
            pl.BlockSpec((1, n), lambda i: (0, 0)),
            pl.BlockSpec((1, ROWS), lambda i: (0, i)),
            pl.BlockSpec((1, n), lambda i: (0, 0)),
        ],
        out_specs=pl.BlockSpec((ROWS, K), lambda i: (i, 0)),
        out_shape=jax.ShapeDtypeStruct((n, K), jnp.int32),
    )(feat, feat, sqcol, sqrow, batch2d, batch2d)


GATHER_WINDOW = 128


def _sc_gather(table, idx_flat):
    num_idx = idx_flat.shape[0]
    h = table.shape[1]
    idx2d = idx_flat.reshape(1, num_idx)
    mesh = plsc.VectorSubcoreMesh(core_axis_name="core",
                                  subcore_axis_name="subcore")

    @pl.kernel(out_type=jax.ShapeDtypeStruct((num_idx, h), table.dtype),
               mesh=mesh)
    def gather_kernel(t_hbm, i_hbm, o_hbm):
        def body(i_vmem, o_vmem):
            pltpu.sync_copy(t_hbm.at[i_vmem.at[0]], o_vmem)

        pltpu.emit_pipeline(
            body,
            grid=(num_idx // GATHER_WINDOW,),
            in_specs=[pl.BlockSpec((1, GATHER_WINDOW),
                                   index_map=lambda i: (0, i))],
            out_specs=[pl.BlockSpec((GATHER_WINDOW, h),
                                    index_map=lambda i: (i, 0))],
            core_axis_name=("core", "subcore"),
            dimension_semantics=(pltpu.PARALLEL,),
        )(i_hbm, o_hbm)

    return gather_kernel(table, idx2d)


def _edge_body(feat_r_ref, xj_ref, w1t_ref, b1_ref, g1_ref, be1_ref,
               w2t_ref, b2_ref, g2_ref, be2_ref, out_ref):
    rows = feat_r_ref.shape[0]
    d = feat_r_ref.shape[1]
    h1 = w1t_ref.shape[1]
    h2 = out_ref.shape[1]
    xi = feat_r_ref[...]
    xj = xj_ref[...][:, :d].reshape(rows, K, d)
    xi3 = jnp.broadcast_to(xi[:, None, :], (rows, K, d))
    e = jnp.concatenate([xi3, xj - xi3], axis=2).reshape(rows * K, 2 * d)
    z1 = _bdot(e, w1t_ref[...]) + b1_ref[...]
    a1 = jax.nn.relu(_ln(z1, g1_ref[0, :], be1_ref[0, :]))
    z2 = _bdot(a1, w2t_ref[...]) + b2_ref[...]
    a2 = jax.nn.relu(_ln(z2, g2_ref[0, :], be2_ref[0, :]))
    out_ref[...] = jnp.max(a2.reshape(rows, K, h2), axis=1)


def _edge_mlp(feat, xjg, w1t, b1, g1, be1, w2t, b2, g2, be2):
    n, d = feat.shape
    hp = xjg.shape[1]
    h1 = w1t.shape[1]
    h2 = w2t.shape[1]
    return pl.pallas_call(
        _edge_body,
        grid=(n // ROWS,),
        in_specs=[
            pl.BlockSpec((ROWS, d), lambda i: (i, 0)),
            pl.BlockSpec((ROWS * K, hp), lambda i: (i, 0)),
            pl.BlockSpec((2 * d, h1), lambda i: (0, 0)),
            pl.BlockSpec((1, h1), lambda i: (0, 0)),
            pl.BlockSpec((1, h1), lambda i: (0, 0)),
            pl.BlockSpec((1, h1), lambda i: (0, 0)),
            pl.BlockSpec((h1, h2), lambda i: (0, 0)),
            pl.BlockSpec((1, h2), lambda i: (0, 0)),
            pl.BlockSpec((1, h2), lambda i: (0, 0)),
            pl.BlockSpec((1, h2), lambda i: (0, 0)),
        ],
        out_specs=pl.BlockSpec((ROWS, h2), lambda i: (i, 0)),
        out_shape=jax.ShapeDtypeStruct((n, h2), jnp.float32),
    )(feat, xjg, w1t, b1, g1, be1, w2t, b2, g2, be2)


def _edge_conv(feat, batch2d, p):
    n, d = feat.shape
    sq = jnp.sum(feat * feat, axis=1)
    idx = _knn(feat, sq.reshape(n, 1), sq.reshape(1, n), batch2d)
    hp = ((d + 127) // 128) * 128
    table = jnp.pad(feat, ((0, 0), (0, hp - d)))
    xjg = _sc_gather(table, idx.reshape(-1))
    h1 = p['l1']['W'].shape[0]
    h2 = p['l2']['W'].shape[0]
    return _edge_mlp(
        feat, xjg,
        p['l1']['W'].T, p['l1']['b'].reshape(1, h1),
        p['l1']['g'].reshape(1, h1), p['l1']['be'].reshape(1, h1),
        p['l2']['W'].T, p['l2']['b'].reshape(1, h2),
        p['l2']['g'].reshape(1, h2), p['l2']['be'].reshape(1, h2))


def _global_body(nb, x1_ref, x2_ref, x3_ref, bcol_ref, w1t_ref, b1_ref,
                 g1_ref, be1_ref, w2t_ref, b2_ref, g2_ref, be2_ref, g_ref):
    bcol = bcol_ref[...]
    rows = []
    for b in range(nb):
        mask = bcol == b
        pieces = []
        for xr in (x1_ref, x2_ref, x3_ref):
            xv = xr[...]
            masked = jnp.where(mask, xv, -jnp.inf)
            pieces.append(jnp.max(masked, axis=0, keepdims=True))
        rows.append(jnp.concatenate(pieces, axis=1))
    pooled = jnp.concatenate(rows, axis=0)
    z1 = _bdot(pooled, w1t_ref[...]) + b1_ref[...]
    a1 = jax.nn.relu(_ln(z1, g1_ref[0, :], be1_ref[0, :]))
    z2 = _bdot(a1, w2t_ref[...]) + b2_ref[...]
    g_ref[...] = jax.nn.relu(_ln(z2, g2_ref[0, :], be2_ref[0, :]))


def _global_mlp(nb, x1, x2, x3, bcol, gp):
    hg1 = gp['l1']['W'].shape[0]
    hg2 = gp['l2']['W'].shape[0]
    args = (x1, x2, x3, bcol, gp['l1']['W'].T, gp['l1']['b'].reshape(1, hg1),
            gp['l1']['g'].reshape(1, hg1), gp['l1']['be'].reshape(1, hg1),
            gp['l2']['W'].T, gp['l2']['b'].reshape(1, hg2),
            gp['l2']['g'].reshape(1, hg2), gp['l2']['be'].reshape(1, hg2))
    return pl.pallas_call(
        functools.partial(_global_body, nb),
        out_specs=pl.BlockSpec((nb, hg2), lambda: (0, 0)),
        out_shape=jax.ShapeDtypeStruct((nb, hg2), jnp.float32),
    )(*args)


def _head_body(nb, x1_ref, x2_ref, x3_ref, g_ref, bcol_ref, ycol_ref,
               w1t_ref, b1_ref, g1_ref, be1_ref,
               w2t_ref, b2_ref, g2_ref, be2_ref,
               w3t_ref, b3_ref, g3_ref, be3_ref,
               wnt_ref, out_ref):
    rows = x1_ref.shape[0]
    bcol = bcol_ref[...]
    g = g_ref[...]
    gf = jnp.zeros((rows, g.shape[1]), jnp.float32)
    for b in range(nb):
        gf = jnp.where(bcol == b, g[b:b + 1, :], gf)
    combined = jnp.concatenate(
        [x1_ref[...], x2_ref[...], x3_ref[...], gf], axis=1)
    z1 = _bdot(combined, w1t_ref[...]) + b1_ref[...]
    a1 = jax.nn.relu(_ln(z1, g1_ref[0, :], be1_ref[0, :]))
    z2 = _bdot(a1, w2t_ref[...]) + b2_ref[...]
    a2 = jax.nn.relu(_ln(z2, g2_ref[0, :], be2_ref[0, :]))
    z3 = _bdot(a2, w3t_ref[...]) + b3_ref[...]
    emb = _ln(z3, g3_ref[0, :], be3_ref[0, :])

    norm = jnp.sqrt(jnp.sum(emb * emb, axis=1, keepdims=True))
    emb_n = emb / jnp.maximum(norm, 1e-12)
    cos = _bdot(emb_n, wnt_ref[...])
    cos = jnp.clip(cos, -1.0, 1.0)
    sin = jnp.sqrt(jnp.maximum(1.0 - cos * cos, 0.0))
    labels = ycol_ref[...] - 1
    cls = jax.lax.broadcasted_iota(jnp.int32, cos.shape, 1)
    cos_m = math.cos(M_MARGIN)
    sin_m = math.sin(M_MARGIN)
    out = jnp.where(cls == labels, cos * cos_m - sin * sin_m, cos)
    out_ref[...] = out * S_SCALE


def _head(nb, x1, x2, x3, g, bcol, ycol, hp, arc_w):
    n = x1.shape[0]
    h1 = hp['l1']['W'].shape[0]
    h2 = hp['l2']['W'].shape[0]
    h3 = hp['l3']['W'].shape[0]
    wn = arc_w / jnp.maximum(
        jnp.linalg.norm(arc_w, axis=1, keepdims=True), 1e-12)
    full = lambda a: pl.BlockSpec(a.shape, lambda i: tuple(0 for _ in a.shape))
    row_tile = lambda w: pl.BlockSpec((ROWS, w), lambda i: (i, 0))
    args = (x1, x2, x3, g, bcol, ycol,
            hp['l1']['W'].T, hp['l1']['b'].reshape(1, h1),
            hp['l1']['g'].reshape(1, h1), hp['l1']['be'].reshape(1, h1),
            hp['l2']['W'].T, hp['l2']['b'].reshape(1, h2),
            hp['l2']['g'].reshape(1, h2), hp['l2']['be'].reshape(1, h2),
            hp['l3']['W'].T, hp['l3']['b'].reshape(1, h3),
            hp['l3']['g'].reshape(1, h3), hp['l3']['be'].reshape(1, h3),
            wn.T)
    in_specs = [row_tile(x1.shape[1]), row_tile(x2.shape[1]),
                row_tile(x3.shape[1]), full(g),
                pl.BlockSpec((ROWS, 1), lambda i: (i, 0)),
                pl.BlockSpec((ROWS, 1), lambda i: (i, 0))]
    in_specs += [full(a) for a in args[6:]]
    return pl.pallas_call(
        functools.partial(_head_body, nb),
        grid=(n // ROWS,),
        in_specs=in_specs,
        out_specs=pl.BlockSpec((ROWS, NUM_CLASSES), lambda i: (i, 0)),
        out_shape=jax.ShapeDtypeStruct((n, NUM_CLASSES), jnp.float32),
    )(*args)


def kernel(x, batch, y, params):
    n = x.shape[0]
    nb = 4
    batch = batch.astype(jnp.int32)
    batch2d = batch.reshape(1, n)
    bcol = batch.reshape(n, 1)
    ycol = y.astype(jnp.int32).reshape(n, 1)

    x1 = _edge_conv(x, batch2d, params['conv1'])
    x2 = _edge_conv(x1, batch2d, params['conv2'])
    x3 = _edge_conv(x2, batch2d, params['conv3'])
    g = _global_mlp(nb, x1, x2, x3, bcol, params['global'])
    return _head(nb, x1, x2, x3, g, bcol, ycol, params['head'],
                 params['arc_w'])
